# baseline (speedup 1.0000x reference)
import functools

import jax
import jax.numpy as jnp
from jax import lax
from jax.experimental import pallas as pl
from jax.experimental.pallas import tpu as pltpu
from jax.experimental.pallas import tpu_sc as plsc

FEATURES = 64
VOCAB = 5
BATCH = 16384
SEQ = 50
TABW = VOCAB * FEATURES

_info = plsc.get_sparse_core_info()
NC, NS, L = _info.num_cores, _info.num_subcores, _info.num_lanes
NW = NC * NS
NI = 512
NIB = BATCH // NI
NCH = SEQ * NIB // NW
KB = 8
IB = 4
NSPL = TABW * L
NSR = NSPL // 128

_mesh = plsc.VectorSubcoreMesh(core_axis_name="c", subcore_axis_name="s")


@functools.partial(
    pl.kernel,
    out_type=jax.ShapeDtypeStruct((SEQ, FEATURES, BATCH), jnp.float32),
    mesh=_mesh,
    scratch_types=[
        pltpu.VMEM((VOCAB, FEATURES), jnp.float32),
        pltpu.VMEM((FEATURES,), jnp.float32),
        pltpu.VMEM((TABW,), jnp.float32),
        pltpu.VMEM_SHARED((TABW,), jnp.float32),
        pltpu.VMEM((NSR, 128), jnp.int32),
        pltpu.VMEM((NSPL,), jnp.float32),
        [pltpu.VMEM((NI,), jnp.int32) for _ in range(2)],
        [pltpu.VMEM((FEATURES, NI), jnp.float32) for _ in range(2)],
        pltpu.SemaphoreType.DMA,
        [pltpu.SemaphoreType.DMA for _ in range(2)],
        [pltpu.SemaphoreType.DMA for _ in range(2)],
    ],
)
def _embed(wt_hbm, b_hbm, xt_hbm, out_hbm,
           wt_v, b_v, tf_v, tf_sh, sidx_v, spl_v, idx_b, out_b,
           gsem, isem_b, osem_b):
    cid = lax.axis_index("c")
    sid = lax.axis_index("s")
    wid = sid * NC + cid

    @pl.when(sid == 0)
    def _build():
        pltpu.sync_copy(wt_hbm, wt_v)
        pltpu.sync_copy(b_hbm, b_v)
        for v in range(VOCAB):
            for q in range(FEATURES // L):
                tf_v[pl.ds(v * FEATURES + q * L, L)] = (
                    wt_v[v, pl.ds(q * L, L)] + b_v[pl.ds(q * L, L)])
        pltpu.sync_copy(tf_v, tf_sh)

    plsc.subcore_barrier()

    def _sb2(g, carry):
        row = g // 8
        col = (g % 8) * L
        sidx_v[row, pl.ds(col, L)] = jnp.full((L,), g, jnp.int32)
        return carry

    lax.fori_loop(0, TABW, _sb2, 0)
    sdescs = [
        pltpu.async_copy(tf_sh.at[sidx_v.at[r]],
                         spl_v.at[pl.ds(r * 128, 128)], gsem)
        for r in range(NSR)
    ]
    for d in sdescs:
        d.wait()

    def chunk_src(cc):
        j = cc // NIB
        i0 = (cc % NIB) * NI
        return j * BATCH + i0

    pltpu.async_copy(xt_hbm.at[pl.ds(chunk_src(wid * NCH), NI)],
                     idx_b[0], isem_b[0])

    def body(t, carry):
        for bf in range(2):
            c = t * 2 + bf
            cc = wid * NCH + c

            @pl.when(c + 1 < NCH)
            def _pf(bf=bf, cc=cc):
                pltpu.async_copy(xt_hbm.at[pl.ds(chunk_src(cc + 1), NI)],
                                 idx_b[1 - bf], isem_b[1 - bf])

            @pl.when(c >= 2)
            def _reclaim(bf=bf):
                pltpu.make_async_copy(
                    out_b[bf], out_hbm.at[0, :, pl.ds(0, NI)],
                    osem_b[bf]).wait()

            pltpu.make_async_copy(
                xt_hbm.at[pl.ds(0, NI)], idx_b[bf], isem_b[bf]).wait()

            def kblock(kb, carry2, bf=bf):
                k0 = kb * KB
                spl = [[spl_v[pl.ds(((v * FEATURES) + k0 + kk) * L, L)]
                        for v in range(VOCAB)] for kk in range(KB)]

                @plsc.parallel_loop(0, NI // L, 1, unroll=IB)
                def iblock(g):
                    x = idx_b[bf][pl.ds(g * L, L)]
                    m1 = x == 1
                    m2 = x == 2
                    m3 = x == 3
                    m4 = x == 4
                    for kk in range(KB):
                        tt = spl[kk]
                        acc = jnp.where(m1, tt[1], tt[0])
                        acc = jnp.where(m2, tt[2], acc)
                        acc = jnp.where(m3, tt[3], acc)
                        acc = jnp.where(m4, tt[4], acc)
                        out_b[bf][k0 + kk, pl.ds(g * L, L)] = acc

                return carry2

            lax.fori_loop(0, FEATURES // KB, kblock, 0)

            j = cc // NIB
            i0 = (cc % NIB) * NI
            pltpu.async_copy(out_b[bf], out_hbm.at[j, :, pl.ds(i0, NI)],
                             osem_b[bf])
        return carry

    lax.fori_loop(0, NCH // 2, body, 0)
    for bf in range(2):
        pltpu.make_async_copy(
            out_b[bf], out_hbm.at[0, :, pl.ds(0, NI)], osem_b[bf]).wait()


def kernel(X, W, b):
    wt = W.T
    xt = X.T.reshape(-1).astype(jnp.int32)
    out = _embed(wt, b, xt)
    return out.transpose(2, 0, 1)


# speedup vs baseline: 1.2638x; 1.2638x over previous
import functools

import jax
import jax.numpy as jnp
from jax import lax
from jax.experimental import pallas as pl
from jax.experimental.pallas import tpu as pltpu
from jax.experimental.pallas import tpu_sc as plsc

FEATURES = 64
VOCAB = 5
BATCH = 16384
SEQ = 50
TABW = VOCAB * FEATURES

_info = plsc.get_sparse_core_info()
NC, NS, L = _info.num_cores, _info.num_subcores, _info.num_lanes
NW = NC * NS
NI = 512
NIB = BATCH // NI
NCH = SEQ * NIB // NW
KB = 8
IB = 2
NSPL = TABW * L
NSR = NSPL // 128

_mesh = plsc.VectorSubcoreMesh(core_axis_name="c", subcore_axis_name="s")


@functools.partial(
    pl.kernel,
    out_type=jax.ShapeDtypeStruct((SEQ, FEATURES, BATCH), jnp.float32),
    mesh=_mesh,
    scratch_types=[
        pltpu.VMEM((VOCAB, FEATURES), jnp.float32),
        pltpu.VMEM((FEATURES,), jnp.float32),
        pltpu.VMEM((TABW,), jnp.float32),
        pltpu.VMEM_SHARED((TABW,), jnp.float32),
        pltpu.VMEM((NSR, 128), jnp.int32),
        pltpu.VMEM((NSPL,), jnp.float32),
        [pltpu.VMEM((NI,), jnp.int32) for _ in range(2)],
        [pltpu.VMEM((FEATURES, NI), jnp.float32) for _ in range(2)],
        pltpu.SemaphoreType.DMA,
        [pltpu.SemaphoreType.DMA for _ in range(2)],
        [pltpu.SemaphoreType.DMA for _ in range(2)],
    ],
)
def _embed(wt_hbm, b_hbm, xt_hbm, out_hbm,
           wt_v, b_v, tf_v, tf_sh, sidx_v, spl_v, idx_b, out_b,
           gsem, isem_b, osem_b):
    cid = lax.axis_index("c")
    sid = lax.axis_index("s")
    wid = sid * NC + cid

    @pl.when(sid == 0)
    def _build():
        pltpu.sync_copy(wt_hbm, wt_v)
        pltpu.sync_copy(b_hbm, b_v)
        for v in range(VOCAB):
            for q in range(FEATURES // L):
                tf_v[pl.ds(v * FEATURES + q * L, L)] = (
                    wt_v[v, pl.ds(q * L, L)] + b_v[pl.ds(q * L, L)])
        pltpu.sync_copy(tf_v, tf_sh)

    plsc.subcore_barrier()

    def _sb2(g, carry):
        row = g // 8
        col = (g % 8) * L
        sidx_v[row, pl.ds(col, L)] = jnp.full((L,), g, jnp.int32)
        return carry

    lax.fori_loop(0, TABW, _sb2, 0)
    sdescs = [
        pltpu.async_copy(tf_sh.at[sidx_v.at[r]],
                         spl_v.at[pl.ds(r * 128, 128)], gsem)
        for r in range(NSR)
    ]
    for d in sdescs:
        d.wait()

    def chunk_src(cc):
        j = cc // NIB
        i0 = (cc % NIB) * NI
        return j * BATCH + i0

    pltpu.async_copy(xt_hbm.at[pl.ds(chunk_src(wid * NCH), NI)],
                     idx_b[0], isem_b[0])

    def body(t, carry):
        for bf in range(2):
            c = t * 2 + bf
            cc = wid * NCH + c

            @pl.when(c + 1 < NCH)
            def _pf(bf=bf, cc=cc):
                pltpu.async_copy(xt_hbm.at[pl.ds(chunk_src(cc + 1), NI)],
                                 idx_b[1 - bf], isem_b[1 - bf])

            @pl.when(c >= 2)
            def _reclaim(bf=bf):
                pltpu.make_async_copy(
                    out_b[bf], out_hbm.at[0, :, pl.ds(0, NI)],
                    osem_b[bf]).wait()

            pltpu.make_async_copy(
                xt_hbm.at[pl.ds(0, NI)], idx_b[bf], isem_b[bf]).wait()

            @plsc.parallel_loop(0, FEATURES // KB, 1)
            def kblock(kb, bf=bf):
                k0 = kb * KB
                spl = [[spl_v[pl.ds(((v * FEATURES) + k0 + kk) * L, L)]
                        for v in range(VOCAB)] for kk in range(KB)]

                @plsc.parallel_loop(0, NI // L, 1, unroll=IB)
                def iblock(g):
                    x = idx_b[bf][pl.ds(g * L, L)]
                    m1 = x == 1
                    m2 = x == 2
                    m3 = x == 3
                    m4 = x == 4
                    for kk in range(KB):
                        tt = spl[kk]
                        acc = jnp.where(m1, tt[1], tt[0])
                        acc = jnp.where(m2, tt[2], acc)
                        acc = jnp.where(m3, tt[3], acc)
                        acc = jnp.where(m4, tt[4], acc)
                        out_b[bf][k0 + kk, pl.ds(g * L, L)] = acc

            j = cc // NIB
            i0 = (cc % NIB) * NI
            pltpu.async_copy(out_b[bf], out_hbm.at[j, :, pl.ds(i0, NI)],
                             osem_b[bf])
        return carry

    lax.fori_loop(0, NCH // 2, body, 0)
    for bf in range(2):
        pltpu.make_async_copy(
            out_b[bf], out_hbm.at[0, :, pl.ds(0, NI)], osem_b[bf]).wait()


def kernel(X, W, b):
    wt = W.T
    xt = X.T.reshape(-1).astype(jnp.int32)
    out = _embed(wt, b, xt)
    return out.transpose(2, 0, 1)


# speedup vs baseline: 1.2683x; 1.0035x over previous
import functools

import jax
import jax.numpy as jnp
from jax import lax
from jax.experimental import pallas as pl
from jax.experimental.pallas import tpu as pltpu
from jax.experimental.pallas import tpu_sc as plsc

FEATURES = 64
VOCAB = 5
BATCH = 16384
SEQ = 50
TABW = VOCAB * FEATURES

_info = plsc.get_sparse_core_info()
NC, NS, L = _info.num_cores, _info.num_subcores, _info.num_lanes
NW = NC * NS
NI = 512
NIB = BATCH // NI
NCH = SEQ * NIB // NW
KB = 8
IB = 2
NSPL = TABW * L
NSR = NSPL // 128

_mesh = plsc.VectorSubcoreMesh(core_axis_name="c", subcore_axis_name="s")


@functools.partial(
    pl.kernel,
    out_type=jax.ShapeDtypeStruct((SEQ, FEATURES, BATCH), jnp.float32),
    mesh=_mesh,
    scratch_types=[
        pltpu.VMEM((VOCAB, FEATURES), jnp.float32),
        pltpu.VMEM((FEATURES,), jnp.float32),
        pltpu.VMEM((TABW,), jnp.float32),
        pltpu.VMEM_SHARED((TABW,), jnp.float32),
        pltpu.VMEM((NSR, 128), jnp.int32),
        pltpu.VMEM((NSPL,), jnp.float32),
        [pltpu.VMEM((NI,), jnp.int32) for _ in range(2)],
        [pltpu.VMEM((FEATURES, NI), jnp.float32) for _ in range(2)],
        pltpu.SemaphoreType.DMA,
        [pltpu.SemaphoreType.DMA for _ in range(2)],
        [pltpu.SemaphoreType.DMA for _ in range(2)],
    ],
)
def _embed(wt_hbm, b_hbm, xt_hbm, out_hbm,
           wt_v, b_v, tf_v, tf_sh, sidx_v, spl_v, idx_b, out_b,
           gsem, isem_b, osem_b):
    cid = lax.axis_index("c")
    sid = lax.axis_index("s")
    wid = sid * NC + cid

    @pl.when(sid == 0)
    def _build():
        pltpu.sync_copy(wt_hbm, wt_v)
        pltpu.sync_copy(b_hbm, b_v)
        for v in range(VOCAB):
            for q in range(FEATURES // L):
                tf_v[pl.ds(v * FEATURES + q * L, L)] = (
                    wt_v[v, pl.ds(q * L, L)] + b_v[pl.ds(q * L, L)])
        pltpu.sync_copy(tf_v, tf_sh)

    def chunk_src(cc):
        j = cc // NIB
        i0 = (cc % NIB) * NI
        return j * BATCH + i0

    pltpu.async_copy(xt_hbm.at[pl.ds(chunk_src(wid * NCH), NI)],
                     idx_b[0], isem_b[0])

    def _sb2(g, carry):
        row = g // 8
        col = (g % 8) * L
        sidx_v[row, pl.ds(col, L)] = jnp.full((L,), g, jnp.int32)
        return carry

    lax.fori_loop(0, TABW, _sb2, 0)

    plsc.subcore_barrier()

    sdescs = [
        pltpu.async_copy(tf_sh.at[sidx_v.at[r]],
                         spl_v.at[pl.ds(r * 128, 128)], gsem)
        for r in range(NSR)
    ]
    for d in sdescs:
        d.wait()

    def body(t, carry):
        for bf in range(2):
            c = t * 2 + bf
            cc = wid * NCH + c

            @pl.when(c + 1 < NCH)
            def _pf(bf=bf, cc=cc):
                pltpu.async_copy(xt_hbm.at[pl.ds(chunk_src(cc + 1), NI)],
                                 idx_b[1 - bf], isem_b[1 - bf])

            @pl.when(c >= 2)
            def _reclaim(bf=bf):
                pltpu.make_async_copy(
                    out_b[bf], out_hbm.at[0, :, pl.ds(0, NI)],
                    osem_b[bf]).wait()

            pltpu.make_async_copy(
                xt_hbm.at[pl.ds(0, NI)], idx_b[bf], isem_b[bf]).wait()

            @plsc.parallel_loop(0, FEATURES // KB, 1)
            def kblock(kb, bf=bf):
                k0 = kb * KB
                spl = [[spl_v[pl.ds(((v * FEATURES) + k0 + kk) * L, L)]
                        for v in range(VOCAB)] for kk in range(KB)]

                @plsc.parallel_loop(0, NI // L, 1, unroll=IB)
                def iblock(g):
                    x = idx_b[bf][pl.ds(g * L, L)]
                    m1 = x == 1
                    m2 = x == 2
                    m3 = x == 3
                    m4 = x == 4
                    for kk in range(KB):
                        tt = spl[kk]
                        acc = jnp.where(m1, tt[1], tt[0])
                        acc = jnp.where(m2, tt[2], acc)
                        acc = jnp.where(m3, tt[3], acc)
                        acc = jnp.where(m4, tt[4], acc)
                        out_b[bf][k0 + kk, pl.ds(g * L, L)] = acc

            j = cc // NIB
            i0 = (cc % NIB) * NI
            pltpu.async_copy(out_b[bf], out_hbm.at[j, :, pl.ds(i0, NI)],
                             osem_b[bf])
        return carry

    lax.fori_loop(0, NCH // 2, body, 0)
    for bf in range(2):
        pltpu.make_async_copy(
            out_b[bf], out_hbm.at[0, :, pl.ds(0, NI)], osem_b[bf]).wait()


def kernel(X, W, b):
    wt = W.T
    xt = X.T.reshape(-1).astype(jnp.int32)
    out = _embed(wt, b, xt)
    return out.transpose(2, 0, 1)
